# Initial kernel scaffold; baseline (speedup 1.0000x reference)
import jax
import jax.numpy as jnp
from jax.experimental import pallas as pl


def kernel(x, edge_index, W1l, b1l, W1r, gamma, beta, W2l, b2l, W2r, Wf, bf):
    raise NotImplementedError("write your pallas kernel here")


# speedup vs baseline: 3.9787x; 3.9787x over previous
import jax
import jax.numpy as jnp
from jax import lax
from jax.experimental import pallas as pl
from jax.experimental.pallas import tpu as pltpu
from jax.experimental.pallas import tpu_sc as plsc

N = 10000
E = 320000
NFEAT = 128
NCLASS = 40

NC, NS = 2, 16
NW = NC * NS
CHUNK = 128
EPW = ((E + NW * CHUNK - 1) // (NW * CHUNK)) * CHUNK
EP = EPW * NW
NCHUNKS = EPW // CHUNK
NP = 10112
ZROWS = NP // NS
ZBLK = ZROWS // CHUNK
ZREM = ZROWS % CHUNK
CW = 128

_f32 = jnp.float32

_MESH = plsc.VectorSubcoreMesh(core_axis_name="c", subcore_axis_name="s",
                               num_cores=NC, num_subcores=NS)


def _sc_agg_body(x_hbm, src_hbm, dst_hbm, agg_out,
                 src_idx, dst_idx, rows, agg_sh, sem):
    cid = lax.axis_index("c")
    sid = lax.axis_index("s")
    wid = sid * NC + cid

    def zrow(i, _):
        for j in range(NFEAT // 16):
            rows[i, pl.ds(j * 16, 16)] = jnp.zeros((16,), _f32)
        return 0
    lax.fori_loop(0, CHUNK, zrow, 0)

    zbase = sid * ZROWS
    for b in range(ZBLK):
        pltpu.sync_copy(rows, agg_sh.at[pl.ds(zbase + b * CHUNK, CHUNK)])
    if ZREM:
        pltpu.sync_copy(rows.at[pl.ds(0, ZREM)],
                        agg_sh.at[pl.ds(zbase + ZBLK * CHUNK, ZREM)])
    plsc.subcore_barrier()

    ebase = wid * EPW

    def chunk_body(c, _):
        off = ebase + c * CHUNK
        pltpu.sync_copy(src_hbm.at[pl.ds(off, CHUNK)], src_idx)
        pltpu.sync_copy(dst_hbm.at[pl.ds(off, CHUNK)], dst_idx)
        pltpu.async_copy(x_hbm.at[src_idx], rows, sem).wait()
        pltpu.sync_copy(rows, agg_sh.at[dst_idx], add=True)
        return 0
    lax.fori_loop(0, NCHUNKS, chunk_body, 0)

    plsc.subcore_barrier()
    pltpu.sync_copy(agg_sh.at[pl.ds(sid * ZROWS, ZROWS)],
                    agg_out.at[pl.ds(cid * NP + sid * ZROWS, ZROWS)])


_sc_agg = pl.kernel(
    _sc_agg_body,
    out_type=jax.ShapeDtypeStruct((NC * NP, NFEAT), _f32),
    mesh=_MESH,
    scratch_types=(
        pltpu.VMEM((CHUNK,), jnp.int32),
        pltpu.VMEM((CHUNK,), jnp.int32),
        pltpu.VMEM((CHUNK, NFEAT), _f32),
        pltpu.VMEM_SHARED((NP, NFEAT), _f32),
        pltpu.SemaphoreType.DMA,
    ),
)


def _sc_cnt_body(dst_hbm, cnt_out, dst_idx, ones, zeros, cnt_sh):
    cid = lax.axis_index("c")
    sid = lax.axis_index("s")
    wid = sid * NC + cid

    def zrow(i, _):
        for j in range(CW // 16):
            ones[i, pl.ds(j * 16, 16)] = jnp.ones((16,), _f32)
            zeros[i, pl.ds(j * 16, 16)] = jnp.zeros((16,), _f32)
        return 0
    lax.fori_loop(0, CHUNK, zrow, 0)

    zbase = sid * ZROWS
    for b in range(ZBLK):
        pltpu.sync_copy(zeros, cnt_sh.at[pl.ds(zbase + b * CHUNK, CHUNK)])
    if ZREM:
        pltpu.sync_copy(zeros.at[pl.ds(0, ZREM)],
                        cnt_sh.at[pl.ds(zbase + ZBLK * CHUNK, ZREM)])
    plsc.subcore_barrier()

    ebase = wid * EPW

    def chunk_body(c, _):
        off = ebase + c * CHUNK
        pltpu.sync_copy(dst_hbm.at[pl.ds(off, CHUNK)], dst_idx)
        pltpu.sync_copy(ones, cnt_sh.at[dst_idx], add=True)
        return 0
    lax.fori_loop(0, NCHUNKS, chunk_body, 0)

    plsc.subcore_barrier()
    pltpu.sync_copy(cnt_sh.at[pl.ds(sid * ZROWS, ZROWS)],
                    cnt_out.at[pl.ds(cid * NP + sid * ZROWS, ZROWS)])


_sc_cnt = pl.kernel(
    _sc_cnt_body,
    out_type=jax.ShapeDtypeStruct((NC * NP, CW), _f32),
    mesh=_MESH,
    scratch_types=(
        pltpu.VMEM((CHUNK,), jnp.int32),
        pltpu.VMEM((CHUNK, CW), _f32),
        pltpu.VMEM((CHUNK, CW), _f32),
        pltpu.VMEM_SHARED((NP, CW), _f32),
    ),
)


def _tc_layer1_body(x_ref, agg_ref, cnt_ref, w1lt, b1l, w1rt, gamma, beta,
                    out_ref):
    agg = agg_ref[0:NP, :] + agg_ref[NP:2 * NP, :]
    cnt = cnt_ref[0:NP, 0:1] + cnt_ref[NP:2 * NP, 0:1]
    mean = agg / jnp.maximum(cnt, 1.0)
    out = (jnp.dot(mean, w1lt[...], preferred_element_type=_f32) + b1l[...]
           + jnp.dot(x_ref[...], w1rt[...], preferred_element_type=_f32))
    nrm = jnp.sqrt(jnp.sum(out * out, axis=1, keepdims=True))
    out = out / jnp.maximum(nrm, 1e-12)
    h = jnp.maximum(out, 0.0)
    mask = lax.broadcasted_iota(jnp.int32, (NP, 1), 0) < N
    h = jnp.where(mask, h, 0.0)
    mu = jnp.sum(h, axis=0, keepdims=True) * (1.0 / N)
    d = jnp.where(mask, h - mu, 0.0)
    var = jnp.sum(d * d, axis=0, keepdims=True) * (1.0 / N)
    hn = (h - mu) / jnp.sqrt(var + 1e-5) * gamma[...] + beta[...]
    out_ref[...] = jnp.where(mask, hn, 0.0)


_tc_layer1 = pl.pallas_call(
    _tc_layer1_body,
    out_shape=jax.ShapeDtypeStruct((NP, NFEAT), _f32),
)


def _tc_layer2_body(h_ref, agg_ref, cnt_ref, w2lt, b2l, w2rt, wft, bf,
                    out_ref):
    agg = agg_ref[0:NP, :] + agg_ref[NP:2 * NP, :]
    cnt = cnt_ref[0:NP, 0:1] + cnt_ref[NP:2 * NP, 0:1]
    mean = agg / jnp.maximum(cnt, 1.0)
    out = (jnp.dot(mean, w2lt[...], preferred_element_type=_f32) + b2l[...]
           + jnp.dot(h_ref[...], w2rt[...], preferred_element_type=_f32))
    nrm = jnp.sqrt(jnp.sum(out * out, axis=1, keepdims=True))
    out = out / jnp.maximum(nrm, 1e-12)
    out_ref[...] = jnp.dot(out, wft[...], preferred_element_type=_f32) + bf[...]


_tc_layer2 = pl.pallas_call(
    _tc_layer2_body,
    out_shape=jax.ShapeDtypeStruct((NP, NCLASS), _f32),
)


def kernel(x, edge_index, W1l, b1l, W1r, gamma, beta, W2l, b2l, W2r, Wf, bf):
    src = jnp.concatenate(
        [edge_index[0], jnp.zeros((EP - E,), jnp.int32)])
    dst = jnp.concatenate(
        [edge_index[1], jnp.full((EP - E,), N, jnp.int32)])
    x_p = jnp.concatenate([x, jnp.zeros((NP - N, NFEAT), _f32)])

    cnt = _sc_cnt(dst)
    agg1 = _sc_agg(x_p, src, dst)
    h = _tc_layer1(x_p, agg1, cnt, W1l.T, b1l[None], W1r.T,
                   gamma[None], beta[None])
    agg2 = _sc_agg(h, src, dst)
    out = _tc_layer2(h, agg2, cnt, W2l.T, b2l[None], W2r.T, Wf.T, bf[None])
    return out[:N]
